# baseline (speedup 1.0000x reference)
import functools

import jax
import jax.numpy as jnp
from jax import lax
from jax.experimental import pallas as pl
from jax.experimental.pallas import tpu as pltpu
from jax.experimental.pallas import tpu_sc as plsc

N = 10000
E = 320000
D = 128

NC = 2
NS = 16
NW = NC * NS
CH = 96
NCH = 105
EPW = NCH * CH
EP = NW * EPW
NP = 10240
NPT = NP // NS

_mesh = plsc.VectorSubcoreMesh(core_axis_name="c", subcore_axis_name="s")


@functools.partial(
    pl.kernel,
    out_type=jax.ShapeDtypeStruct((NC, NP), jnp.float32),
    mesh=_mesh,
    scratch_types=[
        pltpu.VMEM((NCH, CH), jnp.int32),
        pltpu.VMEM((CH,), jnp.float32),
        pltpu.VMEM_SHARED((NP,), jnp.float32),
    ],
)
def _degree_kernel(col_hbm, zeros_hbm, ones_hbm, out_hbm, col_v, ones_v, deg_sh):
    c = lax.axis_index("c")
    s = lax.axis_index("s")
    wid = c * NS + s
    pltpu.sync_copy(col_hbm.at[wid], col_v)
    pltpu.sync_copy(ones_hbm, ones_v)
    pltpu.sync_copy(zeros_hbm.at[pl.ds(s * NPT, NPT)], deg_sh.at[pl.ds(s * NPT, NPT)])
    plsc.subcore_barrier()

    def body(j, carry):
        pltpu.sync_copy(ones_v, deg_sh.at[col_v.at[j]], add=True)
        return carry

    lax.fori_loop(0, NCH, body, 0)
    plsc.subcore_barrier()
    pltpu.sync_copy(deg_sh.at[pl.ds(s * NPT, NPT)], out_hbm.at[c, pl.ds(s * NPT, NPT)])


@functools.partial(
    pl.kernel,
    out_type=jax.ShapeDtypeStruct((NC, NP, D), jnp.float32),
    mesh=_mesh,
    scratch_types=[
        pltpu.VMEM((EPW,), jnp.int32),
        pltpu.VMEM((NCH, CH), jnp.int32),
        pltpu.VMEM((2, CH, D), jnp.float32),
        pltpu.VMEM_SHARED((NP, D), jnp.float32),
        pltpu.SemaphoreType.DMA((2,)),
    ],
)
def _agg_kernel(row_hbm, col_hbm, hs_hbm, zeros_hbm, out_hbm,
                row_v, col_v, vals_v, s_sh, sem):
    c = lax.axis_index("c")
    s = lax.axis_index("s")
    wid = c * NS + s
    pltpu.sync_copy(row_hbm.at[wid], row_v)
    pltpu.sync_copy(col_hbm.at[wid], col_v)
    pltpu.sync_copy(zeros_hbm.at[pl.ds(s * NPT, NPT)], s_sh.at[pl.ds(s * NPT, NPT)])
    plsc.subcore_barrier()

    def body(i, carry):
        j = i - 1

        @pl.when(i < NCH)
        def _():
            p = i & 1
            pltpu.async_copy(hs_hbm.at[row_v.at[pl.ds(i * CH, CH)]],
                             vals_v.at[p], sem.at[p])

        @pl.when(i > 0)
        def _():
            q = j & 1
            pltpu.make_async_copy(
                hs_hbm.at[pl.ds(0, CH)], vals_v.at[q], sem.at[q]).wait()
            pltpu.sync_copy(vals_v.at[q], s_sh.at[col_v.at[j]], add=True)

        return carry

    lax.fori_loop(0, NCH + 1, body, 0)
    plsc.subcore_barrier()
    pltpu.sync_copy(s_sh.at[pl.ds(s * NPT, NPT)], out_hbm.at[c, pl.ds(s * NPT, NPT)])


_RB = 2000


def _prep_body(pdeg_ref, x_ref, dis_ref, hs_ref):
    deg = pdeg_ref[:, 0:1] + pdeg_ref[:, 1:2] + 1.0
    dis = lax.rsqrt(deg)
    dis_ref[...] = dis
    hs_ref[...] = dis * x_ref[...]


_prep = pl.pallas_call(
    _prep_body,
    grid=(N // _RB,),
    in_specs=[
        pl.BlockSpec((_RB, 2), lambda i: (i, 0)),
        pl.BlockSpec((_RB, D), lambda i: (i, 0)),
    ],
    out_specs=[
        pl.BlockSpec((_RB, 1), lambda i: (i, 0)),
        pl.BlockSpec((_RB, D), lambda i: (i, 0)),
    ],
    out_shape=[
        jax.ShapeDtypeStruct((N, 1), jnp.float32),
        jax.ShapeDtypeStruct((N, D), jnp.float32),
    ],
)


def _layer_body(p0_ref, p1_ref, hs_ref, dis_ref, w_ref, b_ref, out_ref, *, act):
    dis = dis_ref[...]
    agg = dis * (p0_ref[0] + p1_ref[0] + hs_ref[...])
    h = lax.dot_general(agg, w_ref[...], (((1,), (0,)), ((), ())),
                        precision=lax.Precision.HIGHEST,
                        preferred_element_type=jnp.float32) + b_ref[...]
    if act:
        out_ref[...] = dis * jnp.maximum(h, 0.0)
    else:
        out_ref[...] = h


def _make_layer(act):
    return pl.pallas_call(
        functools.partial(_layer_body, act=act),
        grid=(N // _RB,),
        in_specs=[
            pl.BlockSpec((1, _RB, D), lambda i: (0, i, 0)),
            pl.BlockSpec((1, _RB, D), lambda i: (1, i, 0)),
            pl.BlockSpec((_RB, D), lambda i: (i, 0)),
            pl.BlockSpec((_RB, 1), lambda i: (i, 0)),
            pl.BlockSpec((D, D), lambda i: (0, 0)),
            pl.BlockSpec((1, D), lambda i: (0, 0)),
        ],
        out_specs=pl.BlockSpec((_RB, D), lambda i: (i, 0)),
        out_shape=jax.ShapeDtypeStruct((N, D), jnp.float32),
    )


_layer_mid = _make_layer(True)
_layer_final = _make_layer(False)


def kernel(x, edge_index, W0, b0, W1, b1):
    ei = edge_index.astype(jnp.int32)
    pad_row = jnp.zeros((EP - E,), jnp.int32)
    pad_col = jnp.full((EP - E,), N, jnp.int32)
    row = jnp.concatenate([ei[0], pad_row]).reshape(NW, EPW)
    col = jnp.concatenate([ei[1], pad_col]).reshape(NW, NCH, CH)
    zeros_nd = jnp.zeros((NP, D), jnp.float32)
    zeros_n = jnp.zeros((NP,), jnp.float32)
    ones_c = jnp.ones((CH,), jnp.float32)
    b0r = b0.reshape(1, D)
    b1r = b1.reshape(1, D)

    pdeg = _degree_kernel(col, zeros_n, ones_c)
    pdeg_t = pdeg[:, :N].T

    dis, hs0 = _prep(pdeg_t, x)
    p0 = _agg_kernel(row, col, hs0, zeros_nd)
    hs1 = _layer_mid(p0, p0, hs0, dis, W0, b0r)
    p1 = _agg_kernel(row, col, hs1, zeros_nd)
    out = _layer_final(p1, p1, hs1, dis, W1, b1r)
    return out


# speedup vs baseline: 1.0004x; 1.0004x over previous
import functools

import jax
import jax.numpy as jnp
from jax import lax
from jax.experimental import pallas as pl
from jax.experimental.pallas import tpu as pltpu
from jax.experimental.pallas import tpu_sc as plsc

N = 10000
E = 320000
D = 128

NC = 2
NS = 16
NW = NC * NS
CH = 96
NCH = 105
EPW = NCH * CH
EP = NW * EPW
NP = 10240
NPT = NP // NS

_mesh = plsc.VectorSubcoreMesh(core_axis_name="c", subcore_axis_name="s")


@functools.partial(
    pl.kernel,
    out_type=jax.ShapeDtypeStruct((NC, NP), jnp.float32),
    mesh=_mesh,
    scratch_types=[
        pltpu.VMEM((NCH, CH), jnp.int32),
        pltpu.VMEM((CH,), jnp.float32),
        pltpu.VMEM_SHARED((NP,), jnp.float32),
    ],
)
def _degree_kernel(col_hbm, zeros_hbm, ones_hbm, out_hbm, col_v, ones_v, deg_sh):
    c = lax.axis_index("c")
    s = lax.axis_index("s")
    wid = c * NS + s
    pltpu.sync_copy(col_hbm.at[wid], col_v)
    pltpu.sync_copy(ones_hbm, ones_v)
    pltpu.sync_copy(zeros_hbm.at[pl.ds(s * NPT, NPT)], deg_sh.at[pl.ds(s * NPT, NPT)])
    plsc.subcore_barrier()

    def body(j, carry):
        pltpu.sync_copy(ones_v, deg_sh.at[col_v.at[j]], add=True)
        return carry

    lax.fori_loop(0, NCH, body, 0)
    plsc.subcore_barrier()
    pltpu.sync_copy(deg_sh.at[pl.ds(s * NPT, NPT)], out_hbm.at[c, pl.ds(s * NPT, NPT)])


@functools.partial(
    pl.kernel,
    out_type=jax.ShapeDtypeStruct((NC, NP, D), jnp.float32),
    mesh=_mesh,
    scratch_types=[
        pltpu.VMEM((EPW,), jnp.int32),
        pltpu.VMEM((NCH, CH), jnp.int32),
        pltpu.VMEM((2, CH, D), jnp.float32),
        pltpu.VMEM_SHARED((NP, D), jnp.float32),
        pltpu.SemaphoreType.DMA((2,)),
    ],
)
def _agg_kernel(row_hbm, col_hbm, hs_hbm, zeros_hbm, out_hbm,
                row_v, col_v, vals_v, s_sh, sem):
    c = lax.axis_index("c")
    s = lax.axis_index("s")
    wid = c * NS + s
    pltpu.sync_copy(row_hbm.at[wid], row_v)
    pltpu.sync_copy(col_hbm.at[wid], col_v)
    pltpu.sync_copy(zeros_hbm.at[pl.ds(s * NPT, NPT)], s_sh.at[pl.ds(s * NPT, NPT)])
    plsc.subcore_barrier()

    def body(i, carry):
        j = i - 1

        @pl.when(i < NCH)
        def _():
            p = i & 1
            pltpu.async_copy(hs_hbm.at[row_v.at[pl.ds(i * CH, CH)]],
                             vals_v.at[p], sem.at[p])

        @pl.when(i > 0)
        def _():
            q = j & 1
            pltpu.make_async_copy(
                hs_hbm.at[pl.ds(0, CH)], vals_v.at[q], sem.at[q]).wait()
            pltpu.sync_copy(vals_v.at[q], s_sh.at[col_v.at[j]], add=True)

        return carry

    lax.fori_loop(0, NCH + 1, body, 0)
    plsc.subcore_barrier()
    pltpu.sync_copy(s_sh.at[pl.ds(s * NPT, NPT)], out_hbm.at[c, pl.ds(s * NPT, NPT)])


_RB = 2000


def _prep_body(pdeg_ref, x_ref, dis_ref, hs_ref):
    deg = pdeg_ref[:, 0:1] + pdeg_ref[:, 1:2] + 1.0
    dis = lax.rsqrt(deg)
    dis_ref[...] = dis
    hs_ref[...] = dis * x_ref[...]


_prep = pl.pallas_call(
    _prep_body,
    grid=(N // _RB,),
    in_specs=[
        pl.BlockSpec((_RB, 2), lambda i: (i, 0)),
        pl.BlockSpec((_RB, D), lambda i: (i, 0)),
    ],
    out_specs=[
        pl.BlockSpec((_RB, 1), lambda i: (i, 0)),
        pl.BlockSpec((_RB, D), lambda i: (i, 0)),
    ],
    out_shape=[
        jax.ShapeDtypeStruct((N, 1), jnp.float32),
        jax.ShapeDtypeStruct((N, D), jnp.float32),
    ],
)


def _layer_body(p0_ref, p1_ref, hs_ref, dis_ref, w_ref, b_ref, out_ref, *, act):
    dis = dis_ref[...]
    agg = dis * (p0_ref[0] + p1_ref[0] + hs_ref[...])
    h = lax.dot_general(agg, w_ref[...], (((1,), (0,)), ((), ())),
                        precision=lax.Precision.HIGHEST,
                        preferred_element_type=jnp.float32) + b_ref[...]
    if act:
        out_ref[...] = dis * jnp.maximum(h, 0.0)
    else:
        out_ref[...] = h


def _make_layer(act):
    return pl.pallas_call(
        functools.partial(_layer_body, act=act),
        grid=(N // _RB,),
        in_specs=[
            pl.BlockSpec((1, _RB, D), lambda i: (0, i, 0)),
            pl.BlockSpec((1, _RB, D), lambda i: (1, i, 0)),
            pl.BlockSpec((_RB, D), lambda i: (i, 0)),
            pl.BlockSpec((_RB, 1), lambda i: (i, 0)),
            pl.BlockSpec((D, D), lambda i: (0, 0)),
            pl.BlockSpec((1, D), lambda i: (0, 0)),
        ],
        out_specs=pl.BlockSpec((_RB, D), lambda i: (i, 0)),
        out_shape=jax.ShapeDtypeStruct((N, D), jnp.float32),
    )


_layer_mid = _make_layer(True)
_layer_final = _make_layer(False)


def kernel(x, edge_index, W0, b0, W1, b1):
    ei = edge_index.astype(jnp.int32)
    pad_row = jnp.zeros((EP - E,), jnp.int32)
    pad_col = N + jnp.arange(EP - E, dtype=jnp.int32) % (NP - N)
    row = jnp.concatenate([ei[0], pad_row]).reshape(NW, EPW)
    col = jnp.concatenate([ei[1], pad_col]).reshape(NW, NCH, CH)
    zeros_nd = jnp.zeros((NP, D), jnp.float32)
    zeros_n = jnp.zeros((NP,), jnp.float32)
    ones_c = jnp.ones((CH,), jnp.float32)
    b0r = b0.reshape(1, D)
    b1r = b1.reshape(1, D)

    pdeg = _degree_kernel(col, zeros_n, ones_c)
    pdeg_t = pdeg[:, :N].T

    dis, hs0 = _prep(pdeg_t, x)
    p0 = _agg_kernel(row, col, hs0, zeros_nd)
    hs1 = _layer_mid(p0, p0, hs0, dis, W0, b0r)
    p1 = _agg_kernel(row, col, hs1, zeros_nd)
    out = _layer_final(p1, p1, hs1, dis, W1, b1r)
    return out


# speedup vs baseline: 1.8829x; 1.8822x over previous
import functools

import jax
import jax.numpy as jnp
from jax import lax
from jax.experimental import pallas as pl
from jax.experimental.pallas import tpu as pltpu
from jax.experimental.pallas import tpu_sc as plsc

N = 10000
E = 320000
D = 128

NC = 2
NS = 16
NW = NC * NS
EPW = E // NW
CH = 80
NCH = EPW // CH
NP = 10240
NPT = NP // NS

_mesh = plsc.VectorSubcoreMesh(core_axis_name="c", subcore_axis_name="s")


@functools.partial(
    pl.kernel,
    out_type=jax.ShapeDtypeStruct((NC, NP), jnp.float32),
    mesh=_mesh,
    scratch_types=[
        pltpu.VMEM((NCH, CH), jnp.int32),
        pltpu.VMEM((CH,), jnp.float32),
        pltpu.VMEM_SHARED((NP,), jnp.float32),
    ],
)
def _degree_kernel(col_hbm, zeros_hbm, ones_hbm, out_hbm, col_v, ones_v, deg_sh):
    c = lax.axis_index("c")
    s = lax.axis_index("s")
    wid = c * NS + s
    pltpu.sync_copy(col_hbm.at[wid], col_v)
    pltpu.sync_copy(ones_hbm, ones_v)
    pltpu.sync_copy(zeros_hbm.at[pl.ds(s * NPT, NPT)], deg_sh.at[pl.ds(s * NPT, NPT)])
    plsc.subcore_barrier()

    def body(j, carry):
        pltpu.sync_copy(ones_v, deg_sh.at[col_v.at[j]], add=True)
        return carry

    lax.fori_loop(0, NCH, body, 0)
    plsc.subcore_barrier()
    pltpu.sync_copy(deg_sh.at[pl.ds(s * NPT, NPT)], out_hbm.at[c, pl.ds(s * NPT, NPT)])


@functools.partial(
    pl.kernel,
    out_type=jax.ShapeDtypeStruct((NC, NP, D), jnp.float32),
    mesh=_mesh,
    scratch_types=[
        pltpu.VMEM((EPW,), jnp.int32),
        pltpu.VMEM((3, CH), jnp.int32),
        pltpu.VMEM((3, CH, D), jnp.float32),
        pltpu.VMEM_SHARED((NP, D), jnp.float32),
        pltpu.SemaphoreType.DMA((3,)),
        pltpu.SemaphoreType.DMA((3,)),
    ],
)
def _agg_kernel(row_hbm, col_hbm, hs_hbm, zeros_hbm, out_hbm,
                row_v, col_v, vals_v, s_sh, sem, csem):
    c = lax.axis_index("c")
    s = lax.axis_index("s")
    wid = c * NS + s
    pltpu.sync_copy(row_hbm.at[wid], row_v)
    pltpu.sync_copy(zeros_hbm.at[pl.ds(s * NPT, NPT)], s_sh.at[pl.ds(s * NPT, NPT)])
    plsc.subcore_barrier()

    def body(i, carry):
        j = i - 2

        @pl.when(i < NCH)
        def _():
            p = lax.rem(i, 3)
            pltpu.async_copy(col_hbm.at[wid, i], col_v.at[p], csem.at[p])
            pltpu.async_copy(hs_hbm.at[row_v.at[pl.ds(i * CH, CH)]],
                             vals_v.at[p], sem.at[p])

        @pl.when(j >= 0)
        def _():
            q = lax.rem(j, 3)
            pltpu.make_async_copy(
                col_hbm.at[wid, 0], col_v.at[q], csem.at[q]).wait()
            pltpu.make_async_copy(
                hs_hbm.at[pl.ds(0, CH)], vals_v.at[q], sem.at[q]).wait()
            pltpu.sync_copy(vals_v.at[q], s_sh.at[col_v.at[q]], add=True)

        return carry

    lax.fori_loop(0, NCH + 2, body, 0)
    plsc.subcore_barrier()
    pltpu.sync_copy(s_sh.at[pl.ds(s * NPT, NPT)], out_hbm.at[c, pl.ds(s * NPT, NPT)])


_RB = 2000


def _prep_body(pdeg_ref, x_ref, dis_ref, hs_ref):
    deg = pdeg_ref[:, 0:1] + pdeg_ref[:, 1:2] + 1.0
    dis = lax.rsqrt(deg)
    dis_ref[...] = dis
    hs_ref[...] = dis * x_ref[...]


_prep = pl.pallas_call(
    _prep_body,
    grid=(N // _RB,),
    in_specs=[
        pl.BlockSpec((_RB, 2), lambda i: (i, 0)),
        pl.BlockSpec((_RB, D), lambda i: (i, 0)),
    ],
    out_specs=[
        pl.BlockSpec((_RB, 1), lambda i: (i, 0)),
        pl.BlockSpec((_RB, D), lambda i: (i, 0)),
    ],
    out_shape=[
        jax.ShapeDtypeStruct((N, 1), jnp.float32),
        jax.ShapeDtypeStruct((N, D), jnp.float32),
    ],
)


def _layer_body(p0_ref, p1_ref, hs_ref, dis_ref, w_ref, b_ref, out_ref, *, act):
    dis = dis_ref[...]
    agg = dis * (p0_ref[0] + p1_ref[0] + hs_ref[...])
    h = lax.dot_general(agg, w_ref[...], (((1,), (0,)), ((), ())),
                        precision=lax.Precision.HIGHEST,
                        preferred_element_type=jnp.float32) + b_ref[...]
    if act:
        out_ref[...] = dis * jnp.maximum(h, 0.0)
    else:
        out_ref[...] = h


def _make_layer(act):
    return pl.pallas_call(
        functools.partial(_layer_body, act=act),
        grid=(N // _RB,),
        in_specs=[
            pl.BlockSpec((1, _RB, D), lambda i: (0, i, 0)),
            pl.BlockSpec((1, _RB, D), lambda i: (1, i, 0)),
            pl.BlockSpec((_RB, D), lambda i: (i, 0)),
            pl.BlockSpec((_RB, 1), lambda i: (i, 0)),
            pl.BlockSpec((D, D), lambda i: (0, 0)),
            pl.BlockSpec((1, D), lambda i: (0, 0)),
        ],
        out_specs=pl.BlockSpec((_RB, D), lambda i: (i, 0)),
        out_shape=jax.ShapeDtypeStruct((N, D), jnp.float32),
    )


_layer_mid = _make_layer(True)
_layer_final = _make_layer(False)


def kernel(x, edge_index, W0, b0, W1, b1):
    row = edge_index[0].astype(jnp.int32).reshape(NW, EPW)
    col = edge_index[1].astype(jnp.int32).reshape(NW, NCH, CH)
    zeros_nd = jnp.zeros((NP, D), jnp.float32)
    zeros_n = jnp.zeros((NP,), jnp.float32)
    ones_c = jnp.ones((CH,), jnp.float32)
    b0r = b0.reshape(1, D)
    b1r = b1.reshape(1, D)

    pdeg = _degree_kernel(col, zeros_n, ones_c)
    pdeg_t = pdeg[:, :N].T

    dis, hs0 = _prep(pdeg_t, x)
    p0 = _agg_kernel(row, col, hs0, zeros_nd)
    hs1 = _layer_mid(p0, p0, hs0, dis, W0, b0r)
    p1 = _agg_kernel(row, col, hs1, zeros_nd)
    out = _layer_final(p1, p1, hs1, dis, W1, b1r)
    return out
